# baseline (speedup 1.0000x reference)
import functools

import jax
import jax.numpy as jnp
from jax import lax
from jax.experimental import pallas as pl
from jax.experimental.pallas import tpu as pltpu
from jax.experimental.pallas import tpu_sc as plsc

D_MODEL = 768
BATCH = 4
SEQ = 4096
NTOK = BATCH * SEQ
NUM_CORES = 2
NUM_SUBCORES = 16
NW = NUM_CORES * NUM_SUBCORES
POS_PER_W = SEQ // NW
PCHUNK = 8
NCHUNK = POS_PER_W // PCHUNK
CROWS = PCHUNK * BATCH
NBUF = 4
AHEAD = 3
LANES = 16
VECS_PER_ROW = D_MODEL // LANES


@functools.partial(
    pl.kernel,
    out_type=jax.ShapeDtypeStruct((NTOK, D_MODEL), jnp.float32),
    mesh=plsc.VectorSubcoreMesh(core_axis_name="c", subcore_axis_name="s"),
    scratch_types=(
        [pltpu.VMEM((NCHUNK * CROWS,), jnp.int32)]
        + [pltpu.VMEM((CROWS, D_MODEL), jnp.float32)] * NBUF
        + [pltpu.VMEM((PCHUNK, D_MODEL), jnp.float32)] * NBUF
        + [pltpu.SemaphoreType.DMA] * (3 * NBUF)
    ),
)
def _embed_sc(x_hbm, table_hbm, pe_hbm, out_hbm, idx_flat, *bufs):
    rows = bufs[0:NBUF]
    pes = bufs[NBUF:2 * NBUF]
    sg = bufs[2 * NBUF:3 * NBUF]
    sp = bufs[3 * NBUF:4 * NBUF]
    so = bufs[4 * NBUF:5 * NBUF]

    wid = lax.axis_index("s") * NUM_CORES + lax.axis_index("c")
    p_base = wid * POS_PER_W

    pltpu.sync_copy(x_hbm.at[wid], idx_flat)

    def issue(c, buf):
        g = pltpu.async_copy(
            table_hbm.at[idx_flat.at[pl.ds(c * CROWS, CROWS)]],
            rows[buf], sg[buf])
        p = pltpu.async_copy(
            pe_hbm.at[pl.ds(p_base + c * PCHUNK, PCHUNK)], pes[buf], sp[buf])
        return g, p

    inflight = [None] * NBUF
    outflight = [None] * NBUF
    for k in range(AHEAD):
        inflight[k] = issue(k, k)

    for c in range(NCHUNK):
        buf = c % NBUF
        a = c + AHEAD
        if a < NCHUNK:
            ab = a % NBUF
            if outflight[ab] is not None:
                for d in outflight[ab]:
                    d.wait()
                outflight[ab] = None
            inflight[ab] = issue(a, ab)
        g, p = inflight[buf]
        g.wait()
        p.wait()

        def add_row(i, carry):
            @plsc.parallel_loop(0, VECS_PER_ROW, unroll=1)
            def add_vec(j):
                sl = pl.ds(j * LANES, LANES)
                v = pes[buf][i, sl]
                for b in range(BATCH):
                    plsc.addupdate(rows[buf].at[b * PCHUNK + i, sl], v)

            return carry

        lax.fori_loop(0, PCHUNK, add_row, 0)
        outflight[buf] = [
            pltpu.async_copy(
                rows[buf].at[pl.ds(b * PCHUNK, PCHUNK)],
                out_hbm.at[pl.ds(b * SEQ + p_base + c * PCHUNK, PCHUNK)],
                so[buf])
            for b in range(BATCH)
        ]

    for bufl in outflight:
        if bufl is not None:
            for d in bufl:
                d.wait()


def kernel(x, table, pe):
    xt = (x.astype(jnp.int32)
          .reshape(BATCH, NW, NCHUNK, PCHUNK)
          .transpose(1, 2, 0, 3)
          .reshape(NW, NCHUNK * CROWS))
    out = _embed_sc(xt, table, pe)
    return out.reshape(BATCH, SEQ, D_MODEL)


# speedup vs baseline: 1.0421x; 1.0421x over previous
import functools

import jax
import jax.numpy as jnp
from jax import lax
from jax.experimental import pallas as pl
from jax.experimental.pallas import tpu as pltpu
from jax.experimental.pallas import tpu_sc as plsc

D_MODEL = 768
BATCH = 4
SEQ = 4096
NTOK = BATCH * SEQ
NUM_CORES = 2
NUM_SUBCORES = 16
NW = NUM_CORES * NUM_SUBCORES
POS_PER_W = SEQ // NW
PCHUNK = 8
NCHUNK = POS_PER_W // PCHUNK
CROWS = PCHUNK * BATCH
NBUF = 4
AHEAD = 3
LANES = 16
VECS_PER_ROW = D_MODEL // LANES


@functools.partial(
    pl.kernel,
    out_type=jax.ShapeDtypeStruct((NTOK, D_MODEL), jnp.float32),
    mesh=plsc.VectorSubcoreMesh(core_axis_name="c", subcore_axis_name="s"),
    scratch_types=(
        [pltpu.VMEM((NCHUNK * CROWS,), jnp.int32)]
        + [pltpu.VMEM((CROWS, D_MODEL), jnp.float32)] * NBUF
        + [pltpu.VMEM((PCHUNK, D_MODEL), jnp.float32)] * NBUF
        + [pltpu.SemaphoreType.DMA] * (3 * NBUF)
    ),
)
def _embed_sc(x_hbm, table_hbm, pe_hbm, out_hbm, idx_flat, *bufs):
    rows = bufs[0:NBUF]
    pes = bufs[NBUF:2 * NBUF]
    sg = bufs[2 * NBUF:3 * NBUF]
    sp = bufs[3 * NBUF:4 * NBUF]
    so = bufs[4 * NBUF:5 * NBUF]

    wid = lax.axis_index("s") * NUM_CORES + lax.axis_index("c")
    p_base = wid * POS_PER_W

    pltpu.sync_copy(x_hbm.at[wid], idx_flat)

    def issue(c, buf):
        g = pltpu.async_copy(
            table_hbm.at[idx_flat.at[pl.ds(c * CROWS, CROWS)]],
            rows[buf], sg[buf])
        p = pltpu.async_copy(
            pe_hbm.at[pl.ds(p_base + c * PCHUNK, PCHUNK)], pes[buf], sp[buf])
        return g, p

    inflight = [None] * NBUF
    outflight = [None] * NBUF
    for k in range(AHEAD):
        inflight[k] = issue(k, k)

    for c in range(NCHUNK):
        buf = c % NBUF
        a = c + AHEAD
        if a < NCHUNK:
            ab = a % NBUF
            if outflight[ab] is not None:
                for d in outflight[ab]:
                    d.wait()
                outflight[ab] = None
            inflight[ab] = issue(a, ab)
        g, p = inflight[buf]
        g.wait()
        p.wait()

        def add_row(i, carry):
            @plsc.parallel_loop(0, VECS_PER_ROW, unroll=2)
            def add_vec(j):
                sl = pl.ds(j * LANES, LANES)
                v = pes[buf][i, sl]
                for b in range(BATCH):
                    plsc.addupdate(rows[buf].at[b * PCHUNK + i, sl], v)

            return carry

        lax.fori_loop(0, PCHUNK, add_row, 0)
        outflight[buf] = [
            pltpu.async_copy(
                rows[buf].at[pl.ds(b * PCHUNK, PCHUNK)],
                out_hbm.at[pl.ds(b * SEQ + p_base + c * PCHUNK, PCHUNK)],
                so[buf])
            for b in range(BATCH)
        ]

    for bufl in outflight:
        if bufl is not None:
            for d in bufl:
                d.wait()


def kernel(x, table, pe):
    xt = (x.astype(jnp.int32)
          .reshape(BATCH, NW, NCHUNK, PCHUNK)
          .transpose(1, 2, 0, 3)
          .reshape(NW, NCHUNK * CROWS))
    out = _embed_sc(xt, table, pe)
    return out.reshape(BATCH, SEQ, D_MODEL)
